# baseline (speedup 1.0000x reference)
import functools
import jax
import jax.numpy as jnp
from jax import lax
from jax.experimental import pallas as pl
from jax.experimental.pallas import tpu as pltpu
from jax.experimental.pallas import tpu_sc as plsc

NC, NS = 2, 16
CH0_SPLIT = 32
CHUNK = 128


def _mesh():
    return plsc.VectorSubcoreMesh(core_axis_name="c", subcore_axis_name="s")


def _round_up(v, m):
    return (v + m - 1) // m * m


def _prop(g, src2d, dst2d, zrows, N, ch0=80):
    D = g.shape[1]
    EPR = src2d.shape[0]
    CH0 = ch0
    CH1 = EPR // NS - CH0
    NSP = _round_up(N + 1, NS * CHUNK)
    ZCH = NSP // (NS * CHUNK)
    WBF = NSP // NS
    WBL = N - (NS - 1) * WBF
    assert WBL > 0 and WBF % 8 == 0 and WBL % 8 == 0

    IB = 16
    assert CH0 % IB == 0 and CH1 % IB == 0 and IB % 2 == 0

    @functools.partial(
        pl.kernel,
        out_type=jax.ShapeDtypeStruct((NC, N, D), jnp.float32),
        mesh=_mesh(),
        scratch_types=[
            pltpu.VMEM((IB, CHUNK), jnp.int32),
            pltpu.VMEM((IB, CHUNK), jnp.int32),
            pltpu.VMEM((CHUNK, D), jnp.float32),
            pltpu.VMEM((CHUNK, D), jnp.float32),
            pltpu.VMEM_SHARED((NSP, D), jnp.float32),
            pltpu.SemaphoreType.DMA,
            pltpu.SemaphoreType.DMA,
        ],
    )
    def k(g_hbm, src_hbm, dst_hbm, z_hbm, out_hbm, idxs, idxd, rows0,
          rows1, acc, sem0, sem1):
        c = lax.axis_index("c")
        s = lax.axis_index("s")
        row0 = jnp.where(c == 0, s * CH0, NS * CH0 + s * CH1)
        nblk = jnp.where(c == 0, CH0 // IB, CH1 // IB)
        pltpu.sync_copy(z_hbm, rows0)
        for z in range(ZCH):
            pltpu.sync_copy(
                rows0, acc.at[pl.ds((s * ZCH + z) * CHUNK, CHUNK)])
        plsc.subcore_barrier()

        def blk(bi, carry):
            pltpu.sync_copy(src_hbm.at[pl.ds(row0 + bi * IB, IB)], idxs)
            pltpu.sync_copy(dst_hbm.at[pl.ds(row0 + bi * IB, IB)], idxd)
            cp0 = pltpu.async_copy(g_hbm.at[idxs.at[0]], rows0, sem0)
            for u in range(IB // 2):
                j0, j1 = 2 * u, 2 * u + 1
                cp1 = pltpu.async_copy(g_hbm.at[idxs.at[j1]], rows1, sem1)
                cp0.wait()
                pltpu.sync_copy(rows0, acc.at[idxd.at[j0]], add=True)
                if j1 + 1 < IB:
                    cp0 = pltpu.async_copy(
                        g_hbm.at[idxs.at[j1 + 1]], rows0, sem0)
                cp1.wait()
                pltpu.sync_copy(rows1, acc.at[idxd.at[j1]], add=True)
            return carry

        lax.fori_loop(0, nblk, blk, 0)
        plsc.subcore_barrier()
        base = s * WBF

        @pl.when(s < NS - 1)
        def _():
            pltpu.sync_copy(acc.at[pl.ds(base, WBF)],
                            out_hbm.at[c].at[pl.ds(base, WBF)])

        @pl.when(s == NS - 1)
        def _():
            pltpu.sync_copy(acc.at[pl.ds(base, WBL)],
                            out_hbm.at[c].at[pl.ds(base, WBL)])

    return k(g, src2d, dst2d, zrows)


def _degree(dst2d, ones_rows, zrows, N):
    EPR = dst2d.shape[0]
    CH = EPR // (NC * NS)
    NSP = _round_up(N + 1, NS * CHUNK)
    ZCH = NSP // (NS * CHUNK)
    WBF = NSP // NS
    WBL = N - (NS - 1) * WBF
    assert WBL > 0 and WBF % 8 == 0 and WBL % 8 == 0

    @functools.partial(
        pl.kernel,
        out_type=jax.ShapeDtypeStruct((NC, N, 128), jnp.float32),
        mesh=_mesh(),
        scratch_types=[
            pltpu.VMEM((CH, CHUNK), jnp.int32),
            pltpu.VMEM((CHUNK, 128), jnp.float32),
            pltpu.VMEM_SHARED((NSP, 128), jnp.float32),
        ],
    )
    def k(dst_hbm, ones_hbm, z_hbm, out_hbm, idxd, rows, acc):
        c = lax.axis_index("c")
        s = lax.axis_index("s")
        t = c * NS + s
        pltpu.sync_copy(dst_hbm.at[pl.ds(t * CH, CH)], idxd)
        pltpu.sync_copy(z_hbm, rows)
        for z in range(ZCH):
            pltpu.sync_copy(
                rows, acc.at[pl.ds((s * ZCH + z) * CHUNK, CHUNK)])
        plsc.subcore_barrier()
        pltpu.sync_copy(ones_hbm, rows)

        def body(j, carry):
            pltpu.sync_copy(rows, acc.at[idxd.at[j]], add=True)
            return carry

        lax.fori_loop(0, CH, body, 0)
        plsc.subcore_barrier()
        base = s * WBF

        @pl.when(s < NS - 1)
        def _():
            pltpu.sync_copy(acc.at[pl.ds(base, WBF)],
                            out_hbm.at[c].at[pl.ds(base, WBF)])

        @pl.when(s == NS - 1)
        def _():
            pltpu.sync_copy(acc.at[pl.ds(base, WBL)],
                            out_hbm.at[c].at[pl.ds(base, WBL)])

    return k(dst2d, ones_rows, zrows)


def _dinv_of(cnt0, cnt1):
    return lax.rsqrt(cnt0[:, :1] + cnt1[:, :1] + 1.0)


def _mm_scale(x, W, cnt, bn=1000):
    N, Din = x.shape
    Dout = W.shape[1]

    def body(x_ref, w_ref, cnt_ref, o_ref):
        dinv = _dinv_of(cnt_ref[0], cnt_ref[1])
        o_ref[...] = jnp.dot(x_ref[...], w_ref[...],
                             preferred_element_type=jnp.float32) * dinv

    return pl.pallas_call(
        body,
        grid=(N // bn,),
        in_specs=[
            pl.BlockSpec((bn, Din), lambda i: (i, 0)),
            pl.BlockSpec((Din, Dout), lambda i: (0, 0)),
            pl.BlockSpec((NC, bn, 128), lambda i: (0, i, 0)),
        ],
        out_specs=pl.BlockSpec((bn, Dout), lambda i: (i, 0)),
        out_shape=jax.ShapeDtypeStruct((N, Dout), jnp.float32),
    )(x, W, cnt)


def _fuse_mm(s, g, cnt, b, W, bn=1000):
    N, D = g.shape
    Dout = W.shape[1]

    def body(s_ref, g_ref, cnt_ref, b_ref, w_ref, o_ref):
        dinv = _dinv_of(cnt_ref[0], cnt_ref[1])
        f = jnp.maximum(
            (s_ref[0] + s_ref[1] + g_ref[...]) * dinv + b_ref[...], 0.0)
        o_ref[...] = jnp.dot(f, w_ref[...],
                             preferred_element_type=jnp.float32) * dinv

    return pl.pallas_call(
        body,
        grid=(N // bn,),
        in_specs=[
            pl.BlockSpec((NC, bn, D), lambda i: (0, i, 0)),
            pl.BlockSpec((bn, D), lambda i: (i, 0)),
            pl.BlockSpec((NC, bn, 128), lambda i: (0, i, 0)),
            pl.BlockSpec((1, D), lambda i: (0, 0)),
            pl.BlockSpec((D, Dout), lambda i: (0, 0)),
        ],
        out_specs=pl.BlockSpec((bn, Dout), lambda i: (i, 0)),
        out_shape=jax.ShapeDtypeStruct((N, Dout), jnp.float32),
    )(s, g, cnt, b, W)


def _epilogue(s, g, cnt, b, bn=1000):
    N, D = g.shape

    def body(s_ref, g_ref, cnt_ref, b_ref, o_ref):
        dinv = _dinv_of(cnt_ref[0], cnt_ref[1])
        o_ref[...] = (s_ref[0] + s_ref[1] + g_ref[...]) * dinv + b_ref[...]

    return pl.pallas_call(
        body,
        grid=(N // bn,),
        in_specs=[
            pl.BlockSpec((NC, bn, D), lambda i: (0, i, 0)),
            pl.BlockSpec((bn, D), lambda i: (i, 0)),
            pl.BlockSpec((NC, bn, 128), lambda i: (0, i, 0)),
            pl.BlockSpec((1, D), lambda i: (0, 0)),
        ],
        out_specs=pl.BlockSpec((bn, D), lambda i: (i, 0)),
        out_shape=jax.ShapeDtypeStruct((N, D), jnp.float32),
    )(s, g, cnt, b)


def kernel(x, edge_index, W1, b1, W2, b2, Wmu, bmu, Wls, bls):
    N, _ = x.shape
    E = edge_index.shape[1]
    assert N % NS == 0
    EP = _round_up(E, NC * NS * CHUNK * 8)
    pad = EP - E
    src = jnp.concatenate(
        [edge_index[0], jnp.zeros((pad,), edge_index.dtype)])
    dst = jnp.concatenate(
        [edge_index[1], jnp.full((pad,), N, edge_index.dtype)])
    src2d = src.reshape(EP // CHUNK, CHUNK)
    dst2d = dst.reshape(EP // CHUNK, CHUNK)
    ones128 = jnp.ones((CHUNK, 128), jnp.float32)
    z128 = jnp.zeros((CHUNK, 128), jnp.float32)

    cnt = _degree(dst2d, ones128, z128, N)

    g1 = _mm_scale(x, W1, cnt)
    s1 = _prop(g1, src2d, dst2d, z128, N, ch0=CH0_SPLIT)

    h2 = W2.shape[1]
    W2p = jnp.pad(W2, ((0, 0), (0, 128 - h2)))
    b2p = jnp.pad(b2, (0, 128 - h2))
    g2 = _fuse_mm(s1, g1, cnt, b1.reshape(1, -1), W2p)
    s2 = _prop(g2, src2d, dst2d, z128, N, ch0=CH0_SPLIT)

    Wcat = jnp.concatenate([Wmu, Wls], axis=1)
    Wcatp = jnp.pad(Wcat, ((0, 128 - h2), (0, 0)))
    bcat = jnp.concatenate([bmu, bls]).reshape(1, -1)
    g3 = _fuse_mm(s2, g2, cnt, b2p.reshape(1, -1), Wcatp)
    s3 = _prop(g3, src2d, dst2d, z128, N, ch0=CH0_SPLIT)

    out = _epilogue(s3, g3, cnt, bcat)
    return out[:, :64], out[:, 64:]


# speedup vs baseline: 1.1575x; 1.1575x over previous
import functools
import jax
import jax.numpy as jnp
from jax import lax
from jax.experimental import pallas as pl
from jax.experimental.pallas import tpu as pltpu
from jax.experimental.pallas import tpu_sc as plsc

NC, NS = 2, 16
CH0_SPLIT = 128
CHUNK = 128


def _mesh():
    return plsc.VectorSubcoreMesh(core_axis_name="c", subcore_axis_name="s")


def _round_up(v, m):
    return (v + m - 1) // m * m


def _prop(g, src2d, dst2d, zrows, N, ch0=80):
    D = g.shape[1]
    EPR = src2d.shape[0]
    CH0 = ch0
    CH1 = EPR // NS - CH0
    NSP = _round_up(N + 1, NS * CHUNK)
    ZCH = NSP // (NS * CHUNK)
    WBF = NSP // NS
    WBL = N - (NS - 1) * WBF
    assert WBL > 0 and WBF % 8 == 0 and WBL % 8 == 0

    IB = 16
    assert CH0 % IB == 0 and CH1 % IB == 0 and IB % 2 == 0

    @functools.partial(
        pl.kernel,
        out_type=jax.ShapeDtypeStruct((NC, N, D), jnp.float32),
        mesh=_mesh(),
        scratch_types=[
            pltpu.VMEM((IB, CHUNK), jnp.int32),
            pltpu.VMEM((IB, CHUNK), jnp.int32),
            pltpu.VMEM((CHUNK, D), jnp.float32),
            pltpu.VMEM((CHUNK, D), jnp.float32),
            pltpu.VMEM_SHARED((NSP, D), jnp.float32),
            pltpu.SemaphoreType.DMA,
            pltpu.SemaphoreType.DMA,
        ],
    )
    def k(g_hbm, src_hbm, dst_hbm, z_hbm, out_hbm, idxs, idxd, rows0,
          rows1, acc, sem0, sem1):
        c = lax.axis_index("c")
        s = lax.axis_index("s")
        row0 = jnp.where(c == 0, s * CH0, NS * CH0 + s * CH1)
        nblk = jnp.where(c == 0, CH0 // IB, CH1 // IB)
        pltpu.sync_copy(z_hbm, rows0)
        for z in range(ZCH):
            pltpu.sync_copy(
                rows0, acc.at[pl.ds((s * ZCH + z) * CHUNK, CHUNK)])
        plsc.subcore_barrier()

        def blk(bi, carry):
            pltpu.sync_copy(src_hbm.at[pl.ds(row0 + bi * IB, IB)], idxs)
            pltpu.sync_copy(dst_hbm.at[pl.ds(row0 + bi * IB, IB)], idxd)
            cp0 = pltpu.async_copy(g_hbm.at[idxs.at[0]], rows0, sem0)
            for u in range(IB // 2):
                j0, j1 = 2 * u, 2 * u + 1
                cp1 = pltpu.async_copy(g_hbm.at[idxs.at[j1]], rows1, sem1)
                cp0.wait()
                pltpu.sync_copy(rows0, acc.at[idxd.at[j0]], add=True)
                if j1 + 1 < IB:
                    cp0 = pltpu.async_copy(
                        g_hbm.at[idxs.at[j1 + 1]], rows0, sem0)
                cp1.wait()
                pltpu.sync_copy(rows1, acc.at[idxd.at[j1]], add=True)
            return carry

        lax.fori_loop(0, nblk, blk, 0)
        plsc.subcore_barrier()
        base = s * WBF

        @pl.when(s < NS - 1)
        def _():
            pltpu.sync_copy(acc.at[pl.ds(base, WBF)],
                            out_hbm.at[c].at[pl.ds(base, WBF)])

        @pl.when(s == NS - 1)
        def _():
            pltpu.sync_copy(acc.at[pl.ds(base, WBL)],
                            out_hbm.at[c].at[pl.ds(base, WBL)])

    return k(g, src2d, dst2d, zrows)


def _degree(dst2d, ones_rows, zrows, N):
    EPR = dst2d.shape[0]
    CH = EPR // (NC * NS)
    NSP = _round_up(N + 1, NS * CHUNK)
    ZCH = NSP // (NS * CHUNK)
    WBF = NSP // NS
    WBL = N - (NS - 1) * WBF
    assert WBL > 0 and WBF % 8 == 0 and WBL % 8 == 0

    @functools.partial(
        pl.kernel,
        out_type=jax.ShapeDtypeStruct((NC, N, 128), jnp.float32),
        mesh=_mesh(),
        scratch_types=[
            pltpu.VMEM((CH, CHUNK), jnp.int32),
            pltpu.VMEM((CHUNK, 128), jnp.float32),
            pltpu.VMEM_SHARED((NSP, 128), jnp.float32),
        ],
    )
    def k(dst_hbm, ones_hbm, z_hbm, out_hbm, idxd, rows, acc):
        c = lax.axis_index("c")
        s = lax.axis_index("s")
        t = c * NS + s
        pltpu.sync_copy(dst_hbm.at[pl.ds(t * CH, CH)], idxd)
        pltpu.sync_copy(z_hbm, rows)
        for z in range(ZCH):
            pltpu.sync_copy(
                rows, acc.at[pl.ds((s * ZCH + z) * CHUNK, CHUNK)])
        plsc.subcore_barrier()
        pltpu.sync_copy(ones_hbm, rows)

        def body(j, carry):
            pltpu.sync_copy(rows, acc.at[idxd.at[j]], add=True)
            return carry

        lax.fori_loop(0, CH, body, 0)
        plsc.subcore_barrier()
        base = s * WBF

        @pl.when(s < NS - 1)
        def _():
            pltpu.sync_copy(acc.at[pl.ds(base, WBF)],
                            out_hbm.at[c].at[pl.ds(base, WBF)])

        @pl.when(s == NS - 1)
        def _():
            pltpu.sync_copy(acc.at[pl.ds(base, WBL)],
                            out_hbm.at[c].at[pl.ds(base, WBL)])

    return k(dst2d, ones_rows, zrows)


def _dinv_of(cnt0, cnt1):
    return lax.rsqrt(cnt0[:, :1] + cnt1[:, :1] + 1.0)


def _mm_scale(x, W, cnt, bn=1000):
    N, Din = x.shape
    Dout = W.shape[1]

    def body(x_ref, w_ref, cnt_ref, o_ref):
        dinv = _dinv_of(cnt_ref[0], cnt_ref[1])
        o_ref[...] = jnp.dot(x_ref[...], w_ref[...],
                             preferred_element_type=jnp.float32) * dinv

    return pl.pallas_call(
        body,
        grid=(N // bn,),
        in_specs=[
            pl.BlockSpec((bn, Din), lambda i: (i, 0)),
            pl.BlockSpec((Din, Dout), lambda i: (0, 0)),
            pl.BlockSpec((NC, bn, 128), lambda i: (0, i, 0)),
        ],
        out_specs=pl.BlockSpec((bn, Dout), lambda i: (i, 0)),
        out_shape=jax.ShapeDtypeStruct((N, Dout), jnp.float32),
    )(x, W, cnt)


def _fuse_mm(s, g, cnt, b, W, bn=1000):
    N, D = g.shape
    Dout = W.shape[1]

    def body(s_ref, g_ref, cnt_ref, b_ref, w_ref, o_ref):
        dinv = _dinv_of(cnt_ref[0], cnt_ref[1])
        f = jnp.maximum(
            (s_ref[0] + s_ref[1] + g_ref[...]) * dinv + b_ref[...], 0.0)
        o_ref[...] = jnp.dot(f, w_ref[...],
                             preferred_element_type=jnp.float32) * dinv

    return pl.pallas_call(
        body,
        grid=(N // bn,),
        in_specs=[
            pl.BlockSpec((NC, bn, D), lambda i: (0, i, 0)),
            pl.BlockSpec((bn, D), lambda i: (i, 0)),
            pl.BlockSpec((NC, bn, 128), lambda i: (0, i, 0)),
            pl.BlockSpec((1, D), lambda i: (0, 0)),
            pl.BlockSpec((D, Dout), lambda i: (0, 0)),
        ],
        out_specs=pl.BlockSpec((bn, Dout), lambda i: (i, 0)),
        out_shape=jax.ShapeDtypeStruct((N, Dout), jnp.float32),
    )(s, g, cnt, b, W)


def _epilogue(s, g, cnt, b, bn=1000):
    N, D = g.shape

    def body(s_ref, g_ref, cnt_ref, b_ref, o_ref):
        dinv = _dinv_of(cnt_ref[0], cnt_ref[1])
        o_ref[...] = (s_ref[0] + s_ref[1] + g_ref[...]) * dinv + b_ref[...]

    return pl.pallas_call(
        body,
        grid=(N // bn,),
        in_specs=[
            pl.BlockSpec((NC, bn, D), lambda i: (0, i, 0)),
            pl.BlockSpec((bn, D), lambda i: (i, 0)),
            pl.BlockSpec((NC, bn, 128), lambda i: (0, i, 0)),
            pl.BlockSpec((1, D), lambda i: (0, 0)),
        ],
        out_specs=pl.BlockSpec((bn, D), lambda i: (i, 0)),
        out_shape=jax.ShapeDtypeStruct((N, D), jnp.float32),
    )(s, g, cnt, b)


def kernel(x, edge_index, W1, b1, W2, b2, Wmu, bmu, Wls, bls):
    N, _ = x.shape
    E = edge_index.shape[1]
    assert N % NS == 0
    EP = _round_up(E, NC * NS * CHUNK * 8)
    pad = EP - E
    src = jnp.concatenate(
        [edge_index[0], jnp.zeros((pad,), edge_index.dtype)])
    dst = jnp.concatenate(
        [edge_index[1], jnp.full((pad,), N, edge_index.dtype)])
    src2d = src.reshape(EP // CHUNK, CHUNK)
    dst2d = dst.reshape(EP // CHUNK, CHUNK)
    ones128 = jnp.ones((CHUNK, 128), jnp.float32)
    z128 = jnp.zeros((CHUNK, 128), jnp.float32)

    cnt = _degree(dst2d, ones128, z128, N)

    g1 = _mm_scale(x, W1, cnt)
    s1 = _prop(g1, src2d, dst2d, z128, N, ch0=CH0_SPLIT)

    h2 = W2.shape[1]
    W2p = jnp.pad(W2, ((0, 0), (0, 128 - h2)))
    b2p = jnp.pad(b2, (0, 128 - h2))
    g2 = _fuse_mm(s1, g1, cnt, b1.reshape(1, -1), W2p)
    s2 = _prop(g2, src2d, dst2d, z128, N, ch0=CH0_SPLIT)

    Wcat = jnp.concatenate([Wmu, Wls], axis=1)
    Wcatp = jnp.pad(Wcat, ((0, 128 - h2), (0, 0)))
    bcat = jnp.concatenate([bmu, bls]).reshape(1, -1)
    g3 = _fuse_mm(s2, g2, cnt, b2p.reshape(1, -1), Wcatp)
    s3 = _prop(g3, src2d, dst2d, z128, N, ch0=CH0_SPLIT)

    out = _epilogue(s3, g3, cnt, bcat)
    return out[:, :64], out[:, 64:]


# speedup vs baseline: 3.1992x; 2.7640x over previous
import functools
import jax
import jax.numpy as jnp
from jax import lax
from jax.experimental import pallas as pl
from jax.experimental.pallas import tpu as pltpu
from jax.experimental.pallas import tpu_sc as plsc

NC, NS = 2, 16
CH0_SPLIT = 80
CHUNK = 128


def _mesh():
    return plsc.VectorSubcoreMesh(core_axis_name="c", subcore_axis_name="s")


def _round_up(v, m):
    return (v + m - 1) // m * m


def _prop(g, src2d, dst2d, zrows, N, ch0=80):
    D = g.shape[1]
    EPR = src2d.shape[0]
    CH0 = ch0
    CH1 = EPR // NS - CH0
    NSP = _round_up(N + 1, NS * CHUNK)
    ZCH = NSP // (NS * CHUNK)
    WBF = NSP // NS
    WBL = N - (NS - 1) * WBF
    assert WBL > 0 and WBF % 8 == 0 and WBL % 8 == 0

    IB = 16
    assert CH0 % IB == 0 and CH1 % IB == 0 and IB % 2 == 0

    @functools.partial(
        pl.kernel,
        out_type=jax.ShapeDtypeStruct((NC, N, D), jnp.float32),
        mesh=_mesh(),
        scratch_types=[
            pltpu.VMEM((IB, CHUNK), jnp.int32),
            pltpu.VMEM((IB, CHUNK), jnp.int32),
            pltpu.VMEM((CHUNK, D), jnp.float32),
            pltpu.VMEM((CHUNK, D), jnp.float32),
            pltpu.VMEM_SHARED((NSP, D), jnp.float32),
            pltpu.SemaphoreType.DMA,
            pltpu.SemaphoreType.DMA,
        ],
    )
    def k(g_hbm, src_hbm, dst_hbm, z_hbm, out_hbm, idxs, idxd, rows0,
          rows1, acc, sem0, sem1):
        c = lax.axis_index("c")
        s = lax.axis_index("s")
        row0 = jnp.where(c == 0, s * CH0, NS * CH0 + s * CH1)
        nblk = jnp.where(c == 0, CH0 // IB, CH1 // IB)
        pltpu.sync_copy(z_hbm, rows0)
        for z in range(ZCH):
            pltpu.sync_copy(
                rows0, acc.at[pl.ds((s * ZCH + z) * CHUNK, CHUNK)])
        plsc.subcore_barrier()

        def blk(bi, carry):
            pltpu.sync_copy(src_hbm.at[pl.ds(row0 + bi * IB, IB)], idxs)
            pltpu.sync_copy(dst_hbm.at[pl.ds(row0 + bi * IB, IB)], idxd)
            cp0 = pltpu.async_copy(g_hbm.at[idxs.at[0]], rows0, sem0)
            for u in range(IB // 2):
                j0, j1 = 2 * u, 2 * u + 1
                cp1 = pltpu.async_copy(g_hbm.at[idxs.at[j1]], rows1, sem1)
                cp0.wait()
                pltpu.sync_copy(rows0, acc.at[idxd.at[j0]], add=True)
                if j1 + 1 < IB:
                    cp0 = pltpu.async_copy(
                        g_hbm.at[idxs.at[j1 + 1]], rows0, sem0)
                cp1.wait()
                pltpu.sync_copy(rows1, acc.at[idxd.at[j1]], add=True)
            return carry

        lax.fori_loop(0, nblk, blk, 0)
        plsc.subcore_barrier()
        base = s * WBF

        @pl.when(s < NS - 1)
        def _():
            pltpu.sync_copy(acc.at[pl.ds(base, WBF)],
                            out_hbm.at[c].at[pl.ds(base, WBF)])

        @pl.when(s == NS - 1)
        def _():
            pltpu.sync_copy(acc.at[pl.ds(base, WBL)],
                            out_hbm.at[c].at[pl.ds(base, WBL)])

    return k(g, src2d, dst2d, zrows)


def _degree(dst2d, ones_rows, zrows, N):
    EPR = dst2d.shape[0]
    CH = EPR // (NC * NS)
    NSP = _round_up(N + 1, NS * CHUNK)
    ZCH = NSP // (NS * CHUNK)
    WBF = NSP // NS
    WBL = N - (NS - 1) * WBF
    assert WBL > 0 and WBF % 8 == 0 and WBL % 8 == 0

    @functools.partial(
        pl.kernel,
        out_type=jax.ShapeDtypeStruct((NC, N, 128), jnp.float32),
        mesh=_mesh(),
        scratch_types=[
            pltpu.VMEM((CH, CHUNK), jnp.int32),
            pltpu.VMEM((CHUNK, 128), jnp.float32),
            pltpu.VMEM_SHARED((NSP, 128), jnp.float32),
        ],
    )
    def k(dst_hbm, ones_hbm, z_hbm, out_hbm, idxd, rows, acc):
        c = lax.axis_index("c")
        s = lax.axis_index("s")
        t = c * NS + s
        pltpu.sync_copy(dst_hbm.at[pl.ds(t * CH, CH)], idxd)
        pltpu.sync_copy(z_hbm, rows)
        for z in range(ZCH):
            pltpu.sync_copy(
                rows, acc.at[pl.ds((s * ZCH + z) * CHUNK, CHUNK)])
        plsc.subcore_barrier()
        pltpu.sync_copy(ones_hbm, rows)

        def body(j, carry):
            pltpu.sync_copy(rows, acc.at[idxd.at[j]], add=True)
            return carry

        lax.fori_loop(0, CH, body, 0)
        plsc.subcore_barrier()
        base = s * WBF

        @pl.when(s < NS - 1)
        def _():
            pltpu.sync_copy(acc.at[pl.ds(base, WBF)],
                            out_hbm.at[c].at[pl.ds(base, WBF)])

        @pl.when(s == NS - 1)
        def _():
            pltpu.sync_copy(acc.at[pl.ds(base, WBL)],
                            out_hbm.at[c].at[pl.ds(base, WBL)])

    return k(dst2d, ones_rows, zrows)


def _dinv_of(cnt0, cnt1):
    return lax.rsqrt(cnt0[:, :1] + cnt1[:, :1] + 1.0)


def _mm_scale(x, W, cnt, bn=1000):
    N, Din = x.shape
    Dout = W.shape[1]

    def body(x_ref, w_ref, cnt_ref, o_ref):
        dinv = _dinv_of(cnt_ref[0], cnt_ref[1])
        o_ref[...] = jnp.dot(x_ref[...], w_ref[...],
                             preferred_element_type=jnp.float32) * dinv

    return pl.pallas_call(
        body,
        grid=(N // bn,),
        in_specs=[
            pl.BlockSpec((bn, Din), lambda i: (i, 0)),
            pl.BlockSpec((Din, Dout), lambda i: (0, 0)),
            pl.BlockSpec((NC, bn, 128), lambda i: (0, i, 0)),
        ],
        out_specs=pl.BlockSpec((bn, Dout), lambda i: (i, 0)),
        out_shape=jax.ShapeDtypeStruct((N, Dout), jnp.float32),
    )(x, W, cnt)


def _fuse_mm(s, g, cnt, b, W, bn=1000):
    N, D = g.shape
    Dout = W.shape[1]

    def body(s_ref, g_ref, cnt_ref, b_ref, w_ref, o_ref):
        dinv = _dinv_of(cnt_ref[0], cnt_ref[1])
        f = jnp.maximum(
            (s_ref[0] + s_ref[1] + g_ref[...]) * dinv + b_ref[...], 0.0)
        o_ref[...] = jnp.dot(f, w_ref[...],
                             preferred_element_type=jnp.float32) * dinv

    return pl.pallas_call(
        body,
        grid=(N // bn,),
        in_specs=[
            pl.BlockSpec((NC, bn, D), lambda i: (0, i, 0)),
            pl.BlockSpec((bn, D), lambda i: (i, 0)),
            pl.BlockSpec((NC, bn, 128), lambda i: (0, i, 0)),
            pl.BlockSpec((1, D), lambda i: (0, 0)),
            pl.BlockSpec((D, Dout), lambda i: (0, 0)),
        ],
        out_specs=pl.BlockSpec((bn, Dout), lambda i: (i, 0)),
        out_shape=jax.ShapeDtypeStruct((N, Dout), jnp.float32),
    )(s, g, cnt, b, W)


def _epilogue(s, g, cnt, b, bn=1000):
    N, D = g.shape

    def body(s_ref, g_ref, cnt_ref, b_ref, o_ref):
        dinv = _dinv_of(cnt_ref[0], cnt_ref[1])
        o_ref[...] = (s_ref[0] + s_ref[1] + g_ref[...]) * dinv + b_ref[...]

    return pl.pallas_call(
        body,
        grid=(N // bn,),
        in_specs=[
            pl.BlockSpec((NC, bn, D), lambda i: (0, i, 0)),
            pl.BlockSpec((bn, D), lambda i: (i, 0)),
            pl.BlockSpec((NC, bn, 128), lambda i: (0, i, 0)),
            pl.BlockSpec((1, D), lambda i: (0, 0)),
        ],
        out_specs=pl.BlockSpec((bn, D), lambda i: (i, 0)),
        out_shape=jax.ShapeDtypeStruct((N, D), jnp.float32),
    )(s, g, cnt, b)


def kernel(x, edge_index, W1, b1, W2, b2, Wmu, bmu, Wls, bls):
    N, _ = x.shape
    E = edge_index.shape[1]
    assert N % NS == 0
    EP = _round_up(E, NC * NS * CHUNK * 8)
    pad = EP - E
    src = jnp.concatenate(
        [edge_index[0], jnp.arange(pad, dtype=edge_index.dtype) % N])
    dst = jnp.concatenate(
        [edge_index[1], jnp.full((pad,), N, edge_index.dtype)])
    src2d = src.reshape(EP // CHUNK, CHUNK)
    dst2d = dst.reshape(EP // CHUNK, CHUNK)
    ones128 = jnp.ones((CHUNK, 128), jnp.float32)
    z128 = jnp.zeros((CHUNK, 128), jnp.float32)

    cnt = _degree(dst2d, ones128, z128, N)

    g1 = _mm_scale(x, W1, cnt)
    s1 = _prop(g1, src2d, dst2d, z128, N, ch0=CH0_SPLIT)

    h2 = W2.shape[1]
    W2p = jnp.pad(W2, ((0, 0), (0, 128 - h2)))
    b2p = jnp.pad(b2, (0, 128 - h2))
    g2 = _fuse_mm(s1, g1, cnt, b1.reshape(1, -1), W2p)
    s2 = _prop(g2, src2d, dst2d, z128, N, ch0=CH0_SPLIT)

    Wcat = jnp.concatenate([Wmu, Wls], axis=1)
    Wcatp = jnp.pad(Wcat, ((0, 128 - h2), (0, 0)))
    bcat = jnp.concatenate([bmu, bls]).reshape(1, -1)
    g3 = _fuse_mm(s2, g2, cnt, b2p.reshape(1, -1), Wcatp)
    s3 = _prop(g3, src2d, dst2d, z128, N, ch0=CH0_SPLIT)

    out = _epilogue(s3, g3, cnt, bcat)
    return out[:, :64], out[:, 64:]


# speedup vs baseline: 3.3603x; 1.0504x over previous
import functools
import jax
import jax.numpy as jnp
from jax import lax
from jax.experimental import pallas as pl
from jax.experimental.pallas import tpu as pltpu
from jax.experimental.pallas import tpu_sc as plsc

NC, NS = 2, 16
CH0_SPLIT = 80
CHUNK = 128


def _mesh():
    return plsc.VectorSubcoreMesh(core_axis_name="c", subcore_axis_name="s")


def _round_up(v, m):
    return (v + m - 1) // m * m


def _prop(g, src2d, dst2d, zrows, N, ch0=80):
    D = g.shape[1]
    EPR = src2d.shape[0]
    CH0 = ch0
    CH1 = EPR // NS - CH0
    NSP = _round_up(N + 1, NS * CHUNK)
    ZCH = NSP // (NS * CHUNK)
    WBF = NSP // NS
    WBL = N - (NS - 1) * WBF
    assert WBL > 0 and WBF % 8 == 0 and WBL % 8 == 0

    IB = 40
    assert CH0 % IB == 0 and CH1 % IB == 0 and IB % 2 == 0

    @functools.partial(
        pl.kernel,
        out_type=jax.ShapeDtypeStruct((NC, N, D), jnp.float32),
        mesh=_mesh(),
        scratch_types=[
            pltpu.VMEM((IB, CHUNK), jnp.int32),
            pltpu.VMEM((IB, CHUNK), jnp.int32),
            pltpu.VMEM((CHUNK, D), jnp.float32),
            pltpu.VMEM((CHUNK, D), jnp.float32),
            pltpu.VMEM_SHARED((NSP, D), jnp.float32),
            pltpu.SemaphoreType.DMA,
            pltpu.SemaphoreType.DMA,
            pltpu.SemaphoreType.DMA,
            pltpu.SemaphoreType.DMA,
        ],
    )
    def k(g_hbm, src_hbm, dst_hbm, z_hbm, out_hbm, idxs, idxd, rows0,
          rows1, acc, gsem0, gsem1, ssem0, ssem1):
        c = lax.axis_index("c")
        s = lax.axis_index("s")
        row0 = jnp.where(c == 0, s * CH0, NS * CH0 + s * CH1)
        nblk = jnp.where(c == 0, CH0 // IB, CH1 // IB)
        rows = (rows0, rows1)
        gsem = (gsem0, gsem1)
        ssem = (ssem0, ssem1)
        pltpu.sync_copy(z_hbm, rows0)
        for z in range(ZCH):
            pltpu.sync_copy(
                rows0, acc.at[pl.ds((s * ZCH + z) * CHUNK, CHUNK)])
        plsc.subcore_barrier()

        def blk(bi, carry):
            pltpu.sync_copy(src_hbm.at[pl.ds(row0 + bi * IB, IB)], idxs)
            pltpu.sync_copy(dst_hbm.at[pl.ds(row0 + bi * IB, IB)], idxd)
            gcp = [None, None]
            scp = [None, None]
            gcp[0] = pltpu.async_copy(g_hbm.at[idxs.at[0]], rows0, gsem0)
            gcp[1] = pltpu.async_copy(g_hbm.at[idxs.at[1]], rows1, gsem1)
            for u in range(IB):
                b = u % 2
                gcp[b].wait()
                scp[b] = pltpu.async_copy(
                    rows[b], acc.at[idxd.at[u]], ssem[b], add=True)
                if u + 2 < IB:
                    scp[b].wait()
                    gcp[b] = pltpu.async_copy(
                        g_hbm.at[idxs.at[u + 2]], rows[b], gsem[b])
            scp[(IB - 2) % 2].wait()
            scp[(IB - 1) % 2].wait()
            return carry

        lax.fori_loop(0, nblk, blk, 0)
        plsc.subcore_barrier()
        base = s * WBF

        @pl.when(s < NS - 1)
        def _():
            pltpu.sync_copy(acc.at[pl.ds(base, WBF)],
                            out_hbm.at[c].at[pl.ds(base, WBF)])

        @pl.when(s == NS - 1)
        def _():
            pltpu.sync_copy(acc.at[pl.ds(base, WBL)],
                            out_hbm.at[c].at[pl.ds(base, WBL)])

    return k(g, src2d, dst2d, zrows)


def _degree(dst2d, ones_rows, zrows, N):
    EPR = dst2d.shape[0]
    CH = EPR // (NC * NS)
    NSP = _round_up(N + 1, NS * CHUNK)
    ZCH = NSP // (NS * CHUNK)
    WBF = NSP // NS
    WBL = N - (NS - 1) * WBF
    assert WBL > 0 and WBF % 8 == 0 and WBL % 8 == 0

    @functools.partial(
        pl.kernel,
        out_type=jax.ShapeDtypeStruct((NC, N, 128), jnp.float32),
        mesh=_mesh(),
        scratch_types=[
            pltpu.VMEM((CH, CHUNK), jnp.int32),
            pltpu.VMEM((CHUNK, 128), jnp.float32),
            pltpu.VMEM_SHARED((NSP, 128), jnp.float32),
        ],
    )
    def k(dst_hbm, ones_hbm, z_hbm, out_hbm, idxd, rows, acc):
        c = lax.axis_index("c")
        s = lax.axis_index("s")
        t = c * NS + s
        pltpu.sync_copy(dst_hbm.at[pl.ds(t * CH, CH)], idxd)
        pltpu.sync_copy(z_hbm, rows)
        for z in range(ZCH):
            pltpu.sync_copy(
                rows, acc.at[pl.ds((s * ZCH + z) * CHUNK, CHUNK)])
        plsc.subcore_barrier()
        pltpu.sync_copy(ones_hbm, rows)

        def body(j, carry):
            pltpu.sync_copy(rows, acc.at[idxd.at[j]], add=True)
            return carry

        lax.fori_loop(0, CH, body, 0)
        plsc.subcore_barrier()
        base = s * WBF

        @pl.when(s < NS - 1)
        def _():
            pltpu.sync_copy(acc.at[pl.ds(base, WBF)],
                            out_hbm.at[c].at[pl.ds(base, WBF)])

        @pl.when(s == NS - 1)
        def _():
            pltpu.sync_copy(acc.at[pl.ds(base, WBL)],
                            out_hbm.at[c].at[pl.ds(base, WBL)])

    return k(dst2d, ones_rows, zrows)


def _dinv_of(cnt0, cnt1):
    return lax.rsqrt(cnt0[:, :1] + cnt1[:, :1] + 1.0)


def _mm_scale(x, W, cnt, bn=1000):
    N, Din = x.shape
    Dout = W.shape[1]

    def body(x_ref, w_ref, cnt_ref, o_ref):
        dinv = _dinv_of(cnt_ref[0], cnt_ref[1])
        o_ref[...] = jnp.dot(x_ref[...], w_ref[...],
                             preferred_element_type=jnp.float32) * dinv

    return pl.pallas_call(
        body,
        grid=(N // bn,),
        in_specs=[
            pl.BlockSpec((bn, Din), lambda i: (i, 0)),
            pl.BlockSpec((Din, Dout), lambda i: (0, 0)),
            pl.BlockSpec((NC, bn, 128), lambda i: (0, i, 0)),
        ],
        out_specs=pl.BlockSpec((bn, Dout), lambda i: (i, 0)),
        out_shape=jax.ShapeDtypeStruct((N, Dout), jnp.float32),
    )(x, W, cnt)


def _fuse_mm(s, g, cnt, b, W, bn=1000):
    N, D = g.shape
    Dout = W.shape[1]

    def body(s_ref, g_ref, cnt_ref, b_ref, w_ref, o_ref):
        dinv = _dinv_of(cnt_ref[0], cnt_ref[1])
        f = jnp.maximum(
            (s_ref[0] + s_ref[1] + g_ref[...]) * dinv + b_ref[...], 0.0)
        o_ref[...] = jnp.dot(f, w_ref[...],
                             preferred_element_type=jnp.float32) * dinv

    return pl.pallas_call(
        body,
        grid=(N // bn,),
        in_specs=[
            pl.BlockSpec((NC, bn, D), lambda i: (0, i, 0)),
            pl.BlockSpec((bn, D), lambda i: (i, 0)),
            pl.BlockSpec((NC, bn, 128), lambda i: (0, i, 0)),
            pl.BlockSpec((1, D), lambda i: (0, 0)),
            pl.BlockSpec((D, Dout), lambda i: (0, 0)),
        ],
        out_specs=pl.BlockSpec((bn, Dout), lambda i: (i, 0)),
        out_shape=jax.ShapeDtypeStruct((N, Dout), jnp.float32),
    )(s, g, cnt, b, W)


def _epilogue(s, g, cnt, b, bn=1000):
    N, D = g.shape

    def body(s_ref, g_ref, cnt_ref, b_ref, o_ref):
        dinv = _dinv_of(cnt_ref[0], cnt_ref[1])
        o_ref[...] = (s_ref[0] + s_ref[1] + g_ref[...]) * dinv + b_ref[...]

    return pl.pallas_call(
        body,
        grid=(N // bn,),
        in_specs=[
            pl.BlockSpec((NC, bn, D), lambda i: (0, i, 0)),
            pl.BlockSpec((bn, D), lambda i: (i, 0)),
            pl.BlockSpec((NC, bn, 128), lambda i: (0, i, 0)),
            pl.BlockSpec((1, D), lambda i: (0, 0)),
        ],
        out_specs=pl.BlockSpec((bn, D), lambda i: (i, 0)),
        out_shape=jax.ShapeDtypeStruct((N, D), jnp.float32),
    )(s, g, cnt, b)


def kernel(x, edge_index, W1, b1, W2, b2, Wmu, bmu, Wls, bls):
    N, _ = x.shape
    E = edge_index.shape[1]
    assert N % NS == 0
    EP = _round_up(E, NC * NS * CHUNK * 8)
    pad = EP - E
    src = jnp.concatenate(
        [edge_index[0], jnp.arange(pad, dtype=edge_index.dtype) % N])
    dst = jnp.concatenate(
        [edge_index[1], jnp.full((pad,), N, edge_index.dtype)])
    src2d = src.reshape(EP // CHUNK, CHUNK)
    dst2d = dst.reshape(EP // CHUNK, CHUNK)
    ones128 = jnp.ones((CHUNK, 128), jnp.float32)
    z128 = jnp.zeros((CHUNK, 128), jnp.float32)

    cnt = _degree(dst2d, ones128, z128, N)

    g1 = _mm_scale(x, W1, cnt)
    s1 = _prop(g1, src2d, dst2d, z128, N, ch0=CH0_SPLIT)

    h2 = W2.shape[1]
    W2p = jnp.pad(W2, ((0, 0), (0, 128 - h2)))
    b2p = jnp.pad(b2, (0, 128 - h2))
    g2 = _fuse_mm(s1, g1, cnt, b1.reshape(1, -1), W2p)
    s2 = _prop(g2, src2d, dst2d, z128, N, ch0=CH0_SPLIT)

    Wcat = jnp.concatenate([Wmu, Wls], axis=1)
    Wcatp = jnp.pad(Wcat, ((0, 128 - h2), (0, 0)))
    bcat = jnp.concatenate([bmu, bls]).reshape(1, -1)
    g3 = _fuse_mm(s2, g2, cnt, b2p.reshape(1, -1), Wcatp)
    s3 = _prop(g3, src2d, dst2d, z128, N, ch0=CH0_SPLIT)

    out = _epilogue(s3, g3, cnt, bcat)
    return out[:, :64], out[:, 64:]


# speedup vs baseline: 3.4023x; 1.0125x over previous
import functools
import jax
import jax.numpy as jnp
from jax import lax
from jax.experimental import pallas as pl
from jax.experimental.pallas import tpu as pltpu
from jax.experimental.pallas import tpu_sc as plsc

NC, NS = 2, 16
CH0_SPLIT = 80
CHUNK = 128


def _mesh():
    return plsc.VectorSubcoreMesh(core_axis_name="c", subcore_axis_name="s")


def _round_up(v, m):
    return (v + m - 1) // m * m


def _prop(g, src2d, dst2d, zrows, N, ch0=80):
    D = g.shape[1]
    EPR = src2d.shape[0]
    CH0 = ch0
    CH1 = EPR // NS - CH0
    NSP = _round_up(N + 1, NS * CHUNK)
    ZCH = NSP // (NS * CHUNK)
    WBF = NSP // NS
    WBL = N - (NS - 1) * WBF
    assert WBL > 0 and WBF % 8 == 0 and WBL % 8 == 0

    IB = 40
    assert CH0 % IB == 0 and CH1 % IB == 0 and IB % 2 == 0

    @functools.partial(
        pl.kernel,
        out_type=jax.ShapeDtypeStruct((NC, N, D), jnp.float32),
        mesh=_mesh(),
        scratch_types=[
            pltpu.VMEM((IB, CHUNK), jnp.int32),
            pltpu.VMEM((IB, CHUNK), jnp.int32),
            pltpu.VMEM((CHUNK, D), jnp.float32),
            pltpu.VMEM((CHUNK, D), jnp.float32),
            pltpu.VMEM_SHARED((NSP, D), jnp.float32),
            pltpu.SemaphoreType.DMA,
            pltpu.SemaphoreType.DMA,
            pltpu.SemaphoreType.DMA,
            pltpu.SemaphoreType.DMA,
        ],
    )
    def k(g_hbm, src_hbm, dst_hbm, z_hbm, out_hbm, idxs, idxd, rows0,
          rows1, acc, gsem0, gsem1, ssem0, ssem1):
        c = lax.axis_index("c")
        s = lax.axis_index("s")
        row0 = jnp.where(c == 0, s * CH0, NS * CH0 + s * CH1)
        nblk = jnp.where(c == 0, CH0 // IB, CH1 // IB)
        rows = (rows0, rows1)
        gsem = (gsem0, gsem1)
        ssem = (ssem0, ssem1)
        pltpu.sync_copy(z_hbm, rows0)
        for z in range(ZCH):
            pltpu.sync_copy(
                rows0, acc.at[pl.ds((s * ZCH + z) * CHUNK, CHUNK)])
        plsc.subcore_barrier()

        def blk(bi, carry):
            pltpu.sync_copy(src_hbm.at[pl.ds(row0 + bi * IB, IB)], idxs)
            pltpu.sync_copy(dst_hbm.at[pl.ds(row0 + bi * IB, IB)], idxd)
            gcp = [None, None]
            scp = [None, None]
            gcp[0] = pltpu.async_copy(g_hbm.at[idxs.at[0]], rows0, gsem0)
            gcp[1] = pltpu.async_copy(g_hbm.at[idxs.at[1]], rows1, gsem1)
            for u in range(IB):
                b = u % 2
                gcp[b].wait()
                scp[b] = pltpu.async_copy(
                    rows[b], acc.at[idxd.at[u]], ssem[b], add=True)
                if u + 2 < IB:
                    scp[b].wait()
                    gcp[b] = pltpu.async_copy(
                        g_hbm.at[idxs.at[u + 2]], rows[b], gsem[b])
            scp[(IB - 2) % 2].wait()
            scp[(IB - 1) % 2].wait()
            return carry

        lax.fori_loop(0, nblk, blk, 0)
        plsc.subcore_barrier()
        base = s * WBF

        @pl.when(s < NS - 1)
        def _():
            pltpu.sync_copy(acc.at[pl.ds(base, WBF)],
                            out_hbm.at[c].at[pl.ds(base, WBF)])

        @pl.when(s == NS - 1)
        def _():
            pltpu.sync_copy(acc.at[pl.ds(base, WBL)],
                            out_hbm.at[c].at[pl.ds(base, WBL)])

    return k(g, src2d, dst2d, zrows)


def _degree(dst2d, ones_rows, zrows, N):
    EPR = dst2d.shape[0]
    CH = EPR // (NC * NS)
    NSP = _round_up(N + 1, NS * CHUNK)
    ZCH = NSP // (NS * CHUNK)
    WBF = NSP // NS
    WBL = N - (NS - 1) * WBF
    assert WBL > 0 and WBF % 8 == 0 and WBL % 8 == 0

    @functools.partial(
        pl.kernel,
        out_type=jax.ShapeDtypeStruct((NC, N, 128), jnp.float32),
        mesh=_mesh(),
        scratch_types=[
            pltpu.VMEM((CH, CHUNK), jnp.int32),
            pltpu.VMEM((CHUNK, 128), jnp.float32),
            pltpu.VMEM_SHARED((NSP, 128), jnp.float32),
        ],
    )
    def k(dst_hbm, ones_hbm, z_hbm, out_hbm, idxd, rows, acc):
        c = lax.axis_index("c")
        s = lax.axis_index("s")
        t = c * NS + s
        pltpu.sync_copy(dst_hbm.at[pl.ds(t * CH, CH)], idxd)
        pltpu.sync_copy(z_hbm, rows)
        for z in range(ZCH):
            pltpu.sync_copy(
                rows, acc.at[pl.ds((s * ZCH + z) * CHUNK, CHUNK)])
        plsc.subcore_barrier()
        pltpu.sync_copy(ones_hbm, rows)

        def body(j, carry):
            pltpu.sync_copy(rows, acc.at[idxd.at[j]], add=True)
            return carry

        lax.fori_loop(0, CH, body, 0)
        plsc.subcore_barrier()
        base = s * WBF

        @pl.when(s < NS - 1)
        def _():
            pltpu.sync_copy(acc.at[pl.ds(base, WBF)],
                            out_hbm.at[c].at[pl.ds(base, WBF)])

        @pl.when(s == NS - 1)
        def _():
            pltpu.sync_copy(acc.at[pl.ds(base, WBL)],
                            out_hbm.at[c].at[pl.ds(base, WBL)])

    return k(dst2d, ones_rows, zrows)


def _dinv_of(cnt0, cnt1):
    return lax.rsqrt(cnt0[:, :1] + cnt1[:, :1] + 1.0)


def _mm_scale(x, W, cnt, bn=1000):
    N, Din = x.shape
    Dout = W.shape[1]

    def body(x_ref, w_ref, cnt_ref, o_ref, dinv_ref):
        dinv = _dinv_of(cnt_ref[0], cnt_ref[1])
        dinv_ref[...] = dinv
        o_ref[...] = jnp.dot(x_ref[...], w_ref[...],
                             preferred_element_type=jnp.float32) * dinv

    return pl.pallas_call(
        body,
        grid=(N // bn,),
        in_specs=[
            pl.BlockSpec((bn, Din), lambda i: (i, 0)),
            pl.BlockSpec((Din, Dout), lambda i: (0, 0)),
            pl.BlockSpec((NC, bn, 128), lambda i: (0, i, 0)),
        ],
        out_specs=[
            pl.BlockSpec((bn, Dout), lambda i: (i, 0)),
            pl.BlockSpec((bn, 1), lambda i: (i, 0)),
        ],
        out_shape=[
            jax.ShapeDtypeStruct((N, Dout), jnp.float32),
            jax.ShapeDtypeStruct((N, 1), jnp.float32),
        ],
    )(x, W, cnt)


def _fuse_mm(s, g, dinv, b, W, bn=1000):
    N, D = g.shape
    Dout = W.shape[1]

    def body(s_ref, g_ref, dinv_ref, b_ref, w_ref, o_ref):
        dinv = dinv_ref[...]
        f = jnp.maximum(
            (s_ref[0] + s_ref[1] + g_ref[...]) * dinv + b_ref[...], 0.0)
        o_ref[...] = jnp.dot(f, w_ref[...],
                             preferred_element_type=jnp.float32) * dinv

    return pl.pallas_call(
        body,
        grid=(N // bn,),
        in_specs=[
            pl.BlockSpec((NC, bn, D), lambda i: (0, i, 0)),
            pl.BlockSpec((bn, D), lambda i: (i, 0)),
            pl.BlockSpec((bn, 1), lambda i: (i, 0)),
            pl.BlockSpec((1, D), lambda i: (0, 0)),
            pl.BlockSpec((D, Dout), lambda i: (0, 0)),
        ],
        out_specs=pl.BlockSpec((bn, Dout), lambda i: (i, 0)),
        out_shape=jax.ShapeDtypeStruct((N, Dout), jnp.float32),
    )(s, g, dinv, b, W)


def _epilogue(s, g, dinv, bmu, bls, bn=1000):
    N, D = g.shape
    H = D // 2

    def body(s_ref, g_ref, dinv_ref, bmu_ref, bls_ref, o1_ref, o2_ref):
        res = (s_ref[0] + s_ref[1] + g_ref[...]) * dinv_ref[...]
        o1_ref[...] = res[:, :H] + bmu_ref[...]
        o2_ref[...] = res[:, H:] + bls_ref[...]

    return pl.pallas_call(
        body,
        grid=(N // bn,),
        in_specs=[
            pl.BlockSpec((NC, bn, D), lambda i: (0, i, 0)),
            pl.BlockSpec((bn, D), lambda i: (i, 0)),
            pl.BlockSpec((bn, 1), lambda i: (i, 0)),
            pl.BlockSpec((1, H), lambda i: (0, 0)),
            pl.BlockSpec((1, H), lambda i: (0, 0)),
        ],
        out_specs=[
            pl.BlockSpec((bn, H), lambda i: (i, 0)),
            pl.BlockSpec((bn, H), lambda i: (i, 0)),
        ],
        out_shape=[
            jax.ShapeDtypeStruct((N, H), jnp.float32),
            jax.ShapeDtypeStruct((N, H), jnp.float32),
        ],
    )(s, g, dinv, bmu, bls)


def kernel(x, edge_index, W1, b1, W2, b2, Wmu, bmu, Wls, bls):
    N, _ = x.shape
    E = edge_index.shape[1]
    assert N % NS == 0
    EP = _round_up(E, NC * NS * CHUNK * 8)
    pad = EP - E
    src = jnp.concatenate(
        [edge_index[0], jnp.arange(pad, dtype=edge_index.dtype) % N])
    dst = jnp.concatenate(
        [edge_index[1], jnp.full((pad,), N, edge_index.dtype)])
    src2d = src.reshape(EP // CHUNK, CHUNK)
    dst2d = dst.reshape(EP // CHUNK, CHUNK)
    ones128 = jnp.ones((CHUNK, 128), jnp.float32)
    z128 = jnp.zeros((CHUNK, 128), jnp.float32)

    cnt = _degree(dst2d, ones128, z128, N)

    g1, dinv = _mm_scale(x, W1, cnt)
    s1 = _prop(g1, src2d, dst2d, z128, N, ch0=CH0_SPLIT)

    h2 = W2.shape[1]
    W2p = jnp.pad(W2, ((0, 0), (0, 128 - h2)))
    b2p = jnp.pad(b2, (0, 128 - h2))
    g2 = _fuse_mm(s1, g1, dinv, b1.reshape(1, -1), W2p)
    s2 = _prop(g2, src2d, dst2d, z128, N, ch0=CH0_SPLIT)

    Wcat = jnp.concatenate([Wmu, Wls], axis=1)
    Wcatp = jnp.pad(Wcat, ((0, 128 - h2), (0, 0)))
    g3 = _fuse_mm(s2, g2, dinv, b2p.reshape(1, -1), Wcatp)
    s3 = _prop(g3, src2d, dst2d, z128, N, ch0=CH0_SPLIT)

    mu, logstd = _epilogue(s3, g3, dinv, bmu.reshape(1, -1),
                           bls.reshape(1, -1))
    return mu, logstd


# speedup vs baseline: 3.4856x; 1.0245x over previous
import functools
import jax
import jax.numpy as jnp
from jax import lax
from jax.experimental import pallas as pl
from jax.experimental.pallas import tpu as pltpu
from jax.experimental.pallas import tpu_sc as plsc

NC, NS = 2, 16
CHUNK = 128


def _mesh():
    return plsc.VectorSubcoreMesh(core_axis_name="c", subcore_axis_name="s")


def _round_up(v, m):
    return (v + m - 1) // m * m


def _prop(g, src2d, dst2d, zrows, N):
    D = g.shape[1]
    C = 64
    EPR = src2d.shape[0]
    T = EPR // (NC * NS)
    NSP = _round_up(N + 1, NS * CHUNK)
    ZCH = NSP // (NS * C)
    WBF = NSP // NS
    WBL = N - (NS - 1) * WBF
    assert WBL > 0 and WBF % 8 == 0 and WBL % 8 == 0

    PB = 8
    NB = T // PB
    assert T % PB == 0 and NB >= 3

    @functools.partial(
        pl.kernel,
        out_type=jax.ShapeDtypeStruct((NC, N, D), jnp.float32),
        mesh=_mesh(),
        scratch_types=[
            pltpu.VMEM((3, PB, C), jnp.int32),
            pltpu.VMEM((3, PB, C), jnp.int32),
            [pltpu.VMEM((C, D), jnp.float32) for _ in range(4)],
            pltpu.VMEM_SHARED((NSP, D), jnp.float32),
            [pltpu.SemaphoreType.DMA for _ in range(4)],
            [pltpu.SemaphoreType.DMA for _ in range(4)],
            pltpu.SemaphoreType.DMA,
        ],
    )
    def k(g_hbm, src_hbm, dst_hbm, z_hbm, out_hbm, idxs, idxd, rows, acc,
          gsem, ssem, isem):
        c = lax.axis_index("c")
        s = lax.axis_index("s")
        t = c * NS + s
        base = t * T

        def iload(blk_idx, slot, sync):
            off = base + jnp.where(blk_idx < NB, blk_idx, 0) * PB
            if sync:
                pltpu.sync_copy(src_hbm.at[pl.ds(off, PB)], idxs.at[slot])
                pltpu.sync_copy(dst_hbm.at[pl.ds(off, PB)], idxd.at[slot])
            else:
                pltpu.async_copy(src_hbm.at[pl.ds(off, PB)], idxs.at[slot],
                                 isem)
                pltpu.async_copy(dst_hbm.at[pl.ds(off, PB)], idxd.at[slot],
                                 isem)

        def iwait():
            pltpu.make_async_copy(src_hbm.at[pl.ds(base, PB)],
                                  idxs.at[0], isem).wait()
            pltpu.make_async_copy(dst_hbm.at[pl.ds(base, PB)],
                                  idxd.at[0], isem).wait()

        def gwait(b):
            pltpu.make_async_copy(g_hbm.at[idxs.at[0, 0]], rows[b],
                                  gsem[b]).wait()

        def swait(b):
            pltpu.make_async_copy(rows[b], acc.at[idxd.at[0, 0]],
                                  ssem[b]).wait()

        pltpu.sync_copy(z_hbm.at[pl.ds(0, C)], rows[0])
        for z in range(ZCH):
            pltpu.sync_copy(rows[0],
                            acc.at[pl.ds((s * ZCH + z) * C, C)])
        plsc.subcore_barrier()

        iload(0, 0, True)
        iload(1, 1, False)
        for w in range(3):
            pltpu.async_copy(g_hbm.at[idxs.at[0, w]], rows[w], gsem[w])

        def blk(bi, carry):
            cs = lax.rem(bi, 3)
            cn = lax.rem(bi + 1, 3)
            for kk in range(PB):
                b = kk % 4
                gwait(b)
                pltpu.async_copy(rows[b], acc.at[idxd.at[cs, kk]],
                                 ssem[b], add=True)
                pb = (kk + 3) % 4
                if kk == 0:
                    @pl.when(bi > 0)
                    def _():
                        swait(pb)
                else:
                    swait(pb)
                if kk == 5:
                    iwait()
                if kk < 5:
                    pltpu.async_copy(g_hbm.at[idxs.at[cs, kk + 3]],
                                     rows[pb], gsem[pb])
                else:
                    pltpu.async_copy(g_hbm.at[idxs.at[cn, kk - 5]],
                                     rows[pb], gsem[pb])
            iload(bi + 2, lax.rem(bi + 2, 3), False)
            return carry

        lax.fori_loop(0, NB, blk, 0)
        swait((T - 1) % 4)
        for w in range(3):
            gwait((T + w) % 4)
        iwait()
        plsc.subcore_barrier()
        base = s * WBF

        @pl.when(s < NS - 1)
        def _():
            pltpu.sync_copy(acc.at[pl.ds(base, WBF)],
                            out_hbm.at[c].at[pl.ds(base, WBF)])

        @pl.when(s == NS - 1)
        def _():
            pltpu.sync_copy(acc.at[pl.ds(base, WBL)],
                            out_hbm.at[c].at[pl.ds(base, WBL)])

    return k(g, src2d, dst2d, zrows)


def _degree(dst2d, ones_rows, zrows, N):
    EPR = dst2d.shape[0]
    CH = EPR // (NC * NS)
    NSP = _round_up(N + 1, NS * CHUNK)
    ZCH = NSP // (NS * CHUNK)
    WBF = NSP // NS
    WBL = N - (NS - 1) * WBF
    assert WBL > 0 and WBF % 8 == 0 and WBL % 8 == 0

    @functools.partial(
        pl.kernel,
        out_type=jax.ShapeDtypeStruct((NC, N, 128), jnp.float32),
        mesh=_mesh(),
        scratch_types=[
            pltpu.VMEM((CH, CHUNK), jnp.int32),
            pltpu.VMEM((CHUNK, 128), jnp.float32),
            pltpu.VMEM_SHARED((NSP, 128), jnp.float32),
        ],
    )
    def k(dst_hbm, ones_hbm, z_hbm, out_hbm, idxd, rows, acc):
        c = lax.axis_index("c")
        s = lax.axis_index("s")
        t = c * NS + s
        pltpu.sync_copy(dst_hbm.at[pl.ds(t * CH, CH)], idxd)
        pltpu.sync_copy(z_hbm, rows)
        for z in range(ZCH):
            pltpu.sync_copy(
                rows, acc.at[pl.ds((s * ZCH + z) * CHUNK, CHUNK)])
        plsc.subcore_barrier()
        pltpu.sync_copy(ones_hbm, rows)

        def body(j, carry):
            pltpu.sync_copy(rows, acc.at[idxd.at[j]], add=True)
            return carry

        lax.fori_loop(0, CH, body, 0)
        plsc.subcore_barrier()
        base = s * WBF

        @pl.when(s < NS - 1)
        def _():
            pltpu.sync_copy(acc.at[pl.ds(base, WBF)],
                            out_hbm.at[c].at[pl.ds(base, WBF)])

        @pl.when(s == NS - 1)
        def _():
            pltpu.sync_copy(acc.at[pl.ds(base, WBL)],
                            out_hbm.at[c].at[pl.ds(base, WBL)])

    return k(dst2d, ones_rows, zrows)


def _dinv_of(cnt0, cnt1):
    return lax.rsqrt(cnt0[:, :1] + cnt1[:, :1] + 1.0)


def _mm_scale(x, W, cnt, bn=1000):
    N, Din = x.shape
    Dout = W.shape[1]

    def body(x_ref, w_ref, cnt_ref, o_ref, dinv_ref):
        dinv = _dinv_of(cnt_ref[0], cnt_ref[1])
        dinv_ref[...] = dinv
        o_ref[...] = jnp.dot(x_ref[...], w_ref[...],
                             preferred_element_type=jnp.float32) * dinv

    return pl.pallas_call(
        body,
        grid=(N // bn,),
        in_specs=[
            pl.BlockSpec((bn, Din), lambda i: (i, 0)),
            pl.BlockSpec((Din, Dout), lambda i: (0, 0)),
            pl.BlockSpec((NC, bn, 128), lambda i: (0, i, 0)),
        ],
        out_specs=[
            pl.BlockSpec((bn, Dout), lambda i: (i, 0)),
            pl.BlockSpec((bn, 1), lambda i: (i, 0)),
        ],
        out_shape=[
            jax.ShapeDtypeStruct((N, Dout), jnp.float32),
            jax.ShapeDtypeStruct((N, 1), jnp.float32),
        ],
    )(x, W, cnt)


def _fuse_mm(s, g, dinv, b, W, bn=1000):
    N, D = g.shape
    Dout = W.shape[1]

    def body(s_ref, g_ref, dinv_ref, b_ref, w_ref, o_ref):
        dinv = dinv_ref[...]
        f = jnp.maximum(
            (s_ref[0] + s_ref[1] + g_ref[...]) * dinv + b_ref[...], 0.0)
        o_ref[...] = jnp.dot(f, w_ref[...],
                             preferred_element_type=jnp.float32) * dinv

    return pl.pallas_call(
        body,
        grid=(N // bn,),
        in_specs=[
            pl.BlockSpec((NC, bn, D), lambda i: (0, i, 0)),
            pl.BlockSpec((bn, D), lambda i: (i, 0)),
            pl.BlockSpec((bn, 1), lambda i: (i, 0)),
            pl.BlockSpec((1, D), lambda i: (0, 0)),
            pl.BlockSpec((D, Dout), lambda i: (0, 0)),
        ],
        out_specs=pl.BlockSpec((bn, Dout), lambda i: (i, 0)),
        out_shape=jax.ShapeDtypeStruct((N, Dout), jnp.float32),
    )(s, g, dinv, b, W)


def _epilogue(s, g, dinv, bmu, bls, bn=1000):
    N, D = g.shape
    H = D // 2

    def body(s_ref, g_ref, dinv_ref, bmu_ref, bls_ref, o1_ref, o2_ref):
        res = (s_ref[0] + s_ref[1] + g_ref[...]) * dinv_ref[...]
        o1_ref[...] = res[:, :H] + bmu_ref[...]
        o2_ref[...] = res[:, H:] + bls_ref[...]

    return pl.pallas_call(
        body,
        grid=(N // bn,),
        in_specs=[
            pl.BlockSpec((NC, bn, D), lambda i: (0, i, 0)),
            pl.BlockSpec((bn, D), lambda i: (i, 0)),
            pl.BlockSpec((bn, 1), lambda i: (i, 0)),
            pl.BlockSpec((1, H), lambda i: (0, 0)),
            pl.BlockSpec((1, H), lambda i: (0, 0)),
        ],
        out_specs=[
            pl.BlockSpec((bn, H), lambda i: (i, 0)),
            pl.BlockSpec((bn, H), lambda i: (i, 0)),
        ],
        out_shape=[
            jax.ShapeDtypeStruct((N, H), jnp.float32),
            jax.ShapeDtypeStruct((N, H), jnp.float32),
        ],
    )(s, g, dinv, bmu, bls)


def kernel(x, edge_index, W1, b1, W2, b2, Wmu, bmu, Wls, bls):
    N, _ = x.shape
    E = edge_index.shape[1]
    assert N % NS == 0
    EP = _round_up(E, NC * NS * CHUNK * 8)
    pad = EP - E
    src = jnp.concatenate(
        [edge_index[0], jnp.arange(pad, dtype=edge_index.dtype) % N])
    dst = jnp.concatenate(
        [edge_index[1], jnp.full((pad,), N, edge_index.dtype)])
    src2d = src.reshape(EP // CHUNK, CHUNK)
    dst2d = dst.reshape(EP // CHUNK, CHUNK)
    src64 = src.reshape(EP // 64, 64)
    dst64 = dst.reshape(EP // 64, 64)
    ones128 = jnp.ones((CHUNK, 128), jnp.float32)
    z128 = jnp.zeros((CHUNK, 128), jnp.float32)

    cnt = _degree(dst2d, ones128, z128, N)

    g1, dinv = _mm_scale(x, W1, cnt)
    s1 = _prop(g1, src64, dst64, z128, N)

    h2 = W2.shape[1]
    W2p = jnp.pad(W2, ((0, 0), (0, 128 - h2)))
    b2p = jnp.pad(b2, (0, 128 - h2))
    g2 = _fuse_mm(s1, g1, dinv, b1.reshape(1, -1), W2p)
    s2 = _prop(g2, src64, dst64, z128, N)

    Wcat = jnp.concatenate([Wmu, Wls], axis=1)
    Wcatp = jnp.pad(Wcat, ((0, 128 - h2), (0, 0)))
    g3 = _fuse_mm(s2, g2, dinv, b2p.reshape(1, -1), Wcatp)
    s3 = _prop(g3, src64, dst64, z128, N)

    mu, logstd = _epilogue(s3, g3, dinv, bmu.reshape(1, -1),
                           bls.reshape(1, -1))
    return mu, logstd


# speedup vs baseline: 3.5337x; 1.0138x over previous
import functools
import jax
import jax.numpy as jnp
from jax import lax
from jax.experimental import pallas as pl
from jax.experimental.pallas import tpu as pltpu
from jax.experimental.pallas import tpu_sc as plsc

NC, NS = 2, 16
CHUNK = 128


def _mesh():
    return plsc.VectorSubcoreMesh(core_axis_name="c", subcore_axis_name="s")


def _round_up(v, m):
    return (v + m - 1) // m * m


def _prop(g, src2d, dst2d, zrows, N):
    D = g.shape[1]
    C = 64
    EPR = src2d.shape[0]
    T = EPR // (NC * NS)
    NSP = _round_up(N + 1, NS * CHUNK)
    ZCH = NSP // (NS * C)
    WBF = NSP // NS
    WBL = N - (NS - 1) * WBF
    assert WBL > 0 and WBF % 8 == 0 and WBL % 8 == 0

    PB = 8
    NB = T // PB
    assert T % PB == 0 and NB >= 3

    @functools.partial(
        pl.kernel,
        out_type=jax.ShapeDtypeStruct((NC, N, D), jnp.float32),
        mesh=_mesh(),
        scratch_types=[
            pltpu.VMEM((3, PB, C), jnp.int32),
            pltpu.VMEM((3, PB, C), jnp.int32),
            [pltpu.VMEM((C, D), jnp.float32) for _ in range(4)],
            pltpu.VMEM_SHARED((NSP, D), jnp.float32),
            [pltpu.SemaphoreType.DMA for _ in range(4)],
            [pltpu.SemaphoreType.DMA for _ in range(4)],
            pltpu.SemaphoreType.DMA,
        ],
    )
    def k(g_hbm, src_hbm, dst_hbm, z_hbm, out_hbm, idxs, idxd, rows, acc,
          gsem, ssem, isem):
        c = lax.axis_index("c")
        s = lax.axis_index("s")
        t = c * NS + s
        base = t * T

        def iload(blk_idx, slot, sync):
            off = base + jnp.where(blk_idx < NB, blk_idx, 0) * PB
            if sync:
                pltpu.sync_copy(src_hbm.at[pl.ds(off, PB)], idxs.at[slot])
                pltpu.sync_copy(dst_hbm.at[pl.ds(off, PB)], idxd.at[slot])
            else:
                pltpu.async_copy(src_hbm.at[pl.ds(off, PB)], idxs.at[slot],
                                 isem)
                pltpu.async_copy(dst_hbm.at[pl.ds(off, PB)], idxd.at[slot],
                                 isem)

        def iwait():
            pltpu.make_async_copy(src_hbm.at[pl.ds(base, PB)],
                                  idxs.at[0], isem).wait()
            pltpu.make_async_copy(dst_hbm.at[pl.ds(base, PB)],
                                  idxd.at[0], isem).wait()

        def gwait(b):
            pltpu.make_async_copy(g_hbm.at[idxs.at[0, 0]], rows[b],
                                  gsem[b]).wait()

        def swait(b):
            pltpu.make_async_copy(rows[b], acc.at[idxd.at[0, 0]],
                                  ssem[b]).wait()

        pltpu.sync_copy(z_hbm.at[pl.ds(0, C)], rows[0])
        for z in range(ZCH):
            pltpu.sync_copy(rows[0],
                            acc.at[pl.ds((s * ZCH + z) * C, C)])
        plsc.subcore_barrier()

        iload(0, 0, True)
        iload(1, 1, False)
        for w in range(3):
            pltpu.async_copy(g_hbm.at[idxs.at[0, w]], rows[w], gsem[w])

        def blk(bi, carry):
            cs = lax.rem(bi, 3)
            cn = lax.rem(bi + 1, 3)
            for kk in range(PB):
                b = kk % 4
                gwait(b)
                pltpu.async_copy(rows[b], acc.at[idxd.at[cs, kk]],
                                 ssem[b], add=True)
                pb = (kk + 3) % 4
                if kk == 0:
                    @pl.when(bi > 0)
                    def _():
                        swait(pb)
                else:
                    swait(pb)
                if kk == 5:
                    iwait()
                if kk < 5:
                    pltpu.async_copy(g_hbm.at[idxs.at[cs, kk + 3]],
                                     rows[pb], gsem[pb])
                else:
                    pltpu.async_copy(g_hbm.at[idxs.at[cn, kk - 5]],
                                     rows[pb], gsem[pb])
            iload(bi + 2, lax.rem(bi + 2, 3), False)
            return carry

        lax.fori_loop(0, NB, blk, 0)
        swait((T - 1) % 4)
        for w in range(3):
            gwait((T + w) % 4)
        iwait()
        plsc.subcore_barrier()
        base = s * WBF

        @pl.when(s < NS - 1)
        def _():
            pltpu.sync_copy(acc.at[pl.ds(base, WBF)],
                            out_hbm.at[c].at[pl.ds(base, WBF)])

        @pl.when(s == NS - 1)
        def _():
            pltpu.sync_copy(acc.at[pl.ds(base, WBL)],
                            out_hbm.at[c].at[pl.ds(base, WBL)])

    return k(g, src2d, dst2d, zrows)


def _degree(dst2d, ones_rows, zrows, N):
    EPR = dst2d.shape[0]
    CH = EPR // (NC * NS)
    NSP = _round_up(N + 1, NS * CHUNK)
    ZCH = NSP // (NS * CHUNK)
    WBF = NSP // NS
    WBL = N - (NS - 1) * WBF
    assert WBL > 0 and WBF % 8 == 0 and WBL % 8 == 0

    @functools.partial(
        pl.kernel,
        out_type=jax.ShapeDtypeStruct((NC, N, 128), jnp.float32),
        mesh=_mesh(),
        scratch_types=[
            pltpu.VMEM((CH, CHUNK), jnp.int32),
            pltpu.VMEM((CHUNK, 128), jnp.float32),
            pltpu.VMEM_SHARED((NSP, 128), jnp.float32),
        ],
    )
    def k(dst_hbm, ones_hbm, z_hbm, out_hbm, idxd, rows, acc):
        c = lax.axis_index("c")
        s = lax.axis_index("s")
        t = c * NS + s
        pltpu.sync_copy(dst_hbm.at[pl.ds(t * CH, CH)], idxd)
        pltpu.sync_copy(z_hbm, rows)
        for z in range(ZCH):
            pltpu.sync_copy(
                rows, acc.at[pl.ds((s * ZCH + z) * CHUNK, CHUNK)])
        plsc.subcore_barrier()
        pltpu.sync_copy(ones_hbm, rows)

        def body(j, carry):
            pltpu.sync_copy(rows, acc.at[idxd.at[j]], add=True)
            return carry

        lax.fori_loop(0, CH, body, 0)
        plsc.subcore_barrier()
        base = s * WBF

        @pl.when(s < NS - 1)
        def _():
            pltpu.sync_copy(acc.at[pl.ds(base, WBF)],
                            out_hbm.at[c].at[pl.ds(base, WBF)])

        @pl.when(s == NS - 1)
        def _():
            pltpu.sync_copy(acc.at[pl.ds(base, WBL)],
                            out_hbm.at[c].at[pl.ds(base, WBL)])

    return k(dst2d, ones_rows, zrows)


def _dinv_of(cnt0, cnt1):
    return lax.rsqrt(cnt0[:, :1] + cnt1[:, :1] + 1.0)


def _mm_scale(x, W, cnt, bn=2000):
    N, Din = x.shape
    Dout = W.shape[1]

    def body(x_ref, w_ref, cnt_ref, o_ref, dinv_ref):
        dinv = _dinv_of(cnt_ref[0], cnt_ref[1])
        dinv_ref[...] = dinv
        o_ref[...] = jnp.dot(x_ref[...], w_ref[...],
                             preferred_element_type=jnp.float32) * dinv

    return pl.pallas_call(
        body,
        grid=(N // bn,),
        in_specs=[
            pl.BlockSpec((bn, Din), lambda i: (i, 0)),
            pl.BlockSpec((Din, Dout), lambda i: (0, 0)),
            pl.BlockSpec((NC, bn, 128), lambda i: (0, i, 0)),
        ],
        out_specs=[
            pl.BlockSpec((bn, Dout), lambda i: (i, 0)),
            pl.BlockSpec((bn, 1), lambda i: (i, 0)),
        ],
        out_shape=[
            jax.ShapeDtypeStruct((N, Dout), jnp.float32),
            jax.ShapeDtypeStruct((N, 1), jnp.float32),
        ],
    )(x, W, cnt)


def _fuse_mm(s, g, dinv, b, W, bn=2000):
    N, D = g.shape
    Dout = W.shape[1]

    def body(s_ref, g_ref, dinv_ref, b_ref, w_ref, o_ref):
        dinv = dinv_ref[...]
        f = jnp.maximum(
            (s_ref[0] + s_ref[1] + g_ref[...]) * dinv + b_ref[...], 0.0)
        o_ref[...] = jnp.dot(f, w_ref[...],
                             preferred_element_type=jnp.float32) * dinv

    return pl.pallas_call(
        body,
        grid=(N // bn,),
        in_specs=[
            pl.BlockSpec((NC, bn, D), lambda i: (0, i, 0)),
            pl.BlockSpec((bn, D), lambda i: (i, 0)),
            pl.BlockSpec((bn, 1), lambda i: (i, 0)),
            pl.BlockSpec((1, D), lambda i: (0, 0)),
            pl.BlockSpec((D, Dout), lambda i: (0, 0)),
        ],
        out_specs=pl.BlockSpec((bn, Dout), lambda i: (i, 0)),
        out_shape=jax.ShapeDtypeStruct((N, Dout), jnp.float32),
    )(s, g, dinv, b, W)


def _epilogue(s, g, dinv, bmu, bls, bn=2000):
    N, D = g.shape
    H = D // 2

    def body(s_ref, g_ref, dinv_ref, bmu_ref, bls_ref, o1_ref, o2_ref):
        res = (s_ref[0] + s_ref[1] + g_ref[...]) * dinv_ref[...]
        o1_ref[...] = res[:, :H] + bmu_ref[...]
        o2_ref[...] = res[:, H:] + bls_ref[...]

    return pl.pallas_call(
        body,
        grid=(N // bn,),
        in_specs=[
            pl.BlockSpec((NC, bn, D), lambda i: (0, i, 0)),
            pl.BlockSpec((bn, D), lambda i: (i, 0)),
            pl.BlockSpec((bn, 1), lambda i: (i, 0)),
            pl.BlockSpec((1, H), lambda i: (0, 0)),
            pl.BlockSpec((1, H), lambda i: (0, 0)),
        ],
        out_specs=[
            pl.BlockSpec((bn, H), lambda i: (i, 0)),
            pl.BlockSpec((bn, H), lambda i: (i, 0)),
        ],
        out_shape=[
            jax.ShapeDtypeStruct((N, H), jnp.float32),
            jax.ShapeDtypeStruct((N, H), jnp.float32),
        ],
    )(s, g, dinv, bmu, bls)


def kernel(x, edge_index, W1, b1, W2, b2, Wmu, bmu, Wls, bls):
    N, _ = x.shape
    E = edge_index.shape[1]
    assert N % NS == 0
    EP = _round_up(E, NC * NS * CHUNK * 8)
    pad = EP - E
    src = jnp.concatenate(
        [edge_index[0], jnp.arange(pad, dtype=edge_index.dtype) % N])
    dst = jnp.concatenate(
        [edge_index[1], jnp.full((pad,), N, edge_index.dtype)])
    src2d = src.reshape(EP // CHUNK, CHUNK)
    dst2d = dst.reshape(EP // CHUNK, CHUNK)
    src64 = src.reshape(EP // 64, 64)
    dst64 = dst.reshape(EP // 64, 64)
    ones128 = jnp.ones((CHUNK, 128), jnp.float32)
    z128 = jnp.zeros((CHUNK, 128), jnp.float32)

    cnt = _degree(dst2d, ones128, z128, N)

    g1, dinv = _mm_scale(x, W1, cnt)
    s1 = _prop(g1, src64, dst64, z128, N)

    h2 = W2.shape[1]
    W2p = jnp.pad(W2, ((0, 0), (0, 128 - h2)))
    b2p = jnp.pad(b2, (0, 128 - h2))
    g2 = _fuse_mm(s1, g1, dinv, b1.reshape(1, -1), W2p)
    s2 = _prop(g2, src64, dst64, z128, N)

    Wcat = jnp.concatenate([Wmu, Wls], axis=1)
    Wcatp = jnp.pad(Wcat, ((0, 128 - h2), (0, 0)))
    g3 = _fuse_mm(s2, g2, dinv, b2p.reshape(1, -1), Wcatp)
    s3 = _prop(g3, src64, dst64, z128, N)

    mu, logstd = _epilogue(s3, g3, dinv, bmu.reshape(1, -1),
                           bls.reshape(1, -1))
    return mu, logstd


# speedup vs baseline: 3.5492x; 1.0044x over previous
import functools
import jax
import jax.numpy as jnp
from jax import lax
from jax.experimental import pallas as pl
from jax.experimental.pallas import tpu as pltpu
from jax.experimental.pallas import tpu_sc as plsc

NC, NS = 2, 16
CHUNK = 128


def _mesh():
    return plsc.VectorSubcoreMesh(core_axis_name="c", subcore_axis_name="s")


def _round_up(v, m):
    return (v + m - 1) // m * m


def _prop(g, src2d, dst2d, zrows, N):
    D = g.shape[1]
    C = 64
    EPR = src2d.shape[0]
    T = EPR // (NC * NS)
    NSP = _round_up(N + 1, NS * CHUNK)
    ZCH = NSP // (NS * C)
    WBF = NSP // NS
    WBL = N - (NS - 1) * WBF
    assert WBL > 0 and WBF % 8 == 0 and WBL % 8 == 0

    PB = 16
    NB = T // PB
    assert T % PB == 0 and NB >= 3

    @functools.partial(
        pl.kernel,
        out_type=jax.ShapeDtypeStruct((NC, N, D), jnp.float32),
        mesh=_mesh(),
        scratch_types=[
            pltpu.VMEM((3, PB, C), jnp.int32),
            pltpu.VMEM((3, PB, C), jnp.int32),
            [pltpu.VMEM((C, D), jnp.float32) for _ in range(4)],
            pltpu.VMEM_SHARED((NSP, D), jnp.float32),
            [pltpu.SemaphoreType.DMA for _ in range(4)],
            [pltpu.SemaphoreType.DMA for _ in range(4)],
            pltpu.SemaphoreType.DMA,
        ],
    )
    def k(g_hbm, src_hbm, dst_hbm, z_hbm, out_hbm, idxs, idxd, rows, acc,
          gsem, ssem, isem):
        c = lax.axis_index("c")
        s = lax.axis_index("s")
        t = c * NS + s
        base = t * T

        def iload(blk_idx, slot, sync):
            off = base + jnp.where(blk_idx < NB, blk_idx, 0) * PB
            if sync:
                pltpu.sync_copy(src_hbm.at[pl.ds(off, PB)], idxs.at[slot])
                pltpu.sync_copy(dst_hbm.at[pl.ds(off, PB)], idxd.at[slot])
            else:
                pltpu.async_copy(src_hbm.at[pl.ds(off, PB)], idxs.at[slot],
                                 isem)
                pltpu.async_copy(dst_hbm.at[pl.ds(off, PB)], idxd.at[slot],
                                 isem)

        def iwait():
            pltpu.make_async_copy(src_hbm.at[pl.ds(base, PB)],
                                  idxs.at[0], isem).wait()
            pltpu.make_async_copy(dst_hbm.at[pl.ds(base, PB)],
                                  idxd.at[0], isem).wait()

        def gwait(b):
            pltpu.make_async_copy(g_hbm.at[idxs.at[0, 0]], rows[b],
                                  gsem[b]).wait()

        def swait(b):
            pltpu.make_async_copy(rows[b], acc.at[idxd.at[0, 0]],
                                  ssem[b]).wait()

        pltpu.sync_copy(z_hbm.at[pl.ds(0, C)], rows[0])
        for z in range(ZCH):
            pltpu.sync_copy(rows[0],
                            acc.at[pl.ds((s * ZCH + z) * C, C)])
        plsc.subcore_barrier()

        iload(0, 0, True)
        iload(1, 1, False)
        for w in range(3):
            pltpu.async_copy(g_hbm.at[idxs.at[0, w]], rows[w], gsem[w])

        def blk(bi, carry):
            cs = lax.rem(bi, 3)
            cn = lax.rem(bi + 1, 3)
            for kk in range(PB):
                b = kk % 4
                gwait(b)
                pltpu.async_copy(rows[b], acc.at[idxd.at[cs, kk]],
                                 ssem[b], add=True)
                pb = (kk + 3) % 4
                if kk == 0:
                    @pl.when(bi > 0)
                    def _():
                        swait(pb)
                else:
                    swait(pb)
                if kk == PB - 3:
                    iwait()
                if kk < PB - 3:
                    pltpu.async_copy(g_hbm.at[idxs.at[cs, kk + 3]],
                                     rows[pb], gsem[pb])
                else:
                    pltpu.async_copy(g_hbm.at[idxs.at[cn, kk - (PB - 3)]],
                                     rows[pb], gsem[pb])
            iload(bi + 2, lax.rem(bi + 2, 3), False)
            return carry

        lax.fori_loop(0, NB, blk, 0)
        swait((T - 1) % 4)
        for w in range(3):
            gwait((T + w) % 4)
        iwait()
        plsc.subcore_barrier()
        base = s * WBF

        @pl.when(s < NS - 1)
        def _():
            pltpu.sync_copy(acc.at[pl.ds(base, WBF)],
                            out_hbm.at[c].at[pl.ds(base, WBF)])

        @pl.when(s == NS - 1)
        def _():
            pltpu.sync_copy(acc.at[pl.ds(base, WBL)],
                            out_hbm.at[c].at[pl.ds(base, WBL)])

    return k(g, src2d, dst2d, zrows)


def _degree(dst2d, ones_rows, zrows, N):
    EPR = dst2d.shape[0]
    CH = EPR // (NC * NS)
    NSP = _round_up(N + 1, NS * CHUNK)
    ZCH = NSP // (NS * CHUNK)
    WBF = NSP // NS
    WBL = N - (NS - 1) * WBF
    assert WBL > 0 and WBF % 8 == 0 and WBL % 8 == 0

    @functools.partial(
        pl.kernel,
        out_type=jax.ShapeDtypeStruct((NC, N, 128), jnp.float32),
        mesh=_mesh(),
        scratch_types=[
            pltpu.VMEM((CH, CHUNK), jnp.int32),
            pltpu.VMEM((CHUNK, 128), jnp.float32),
            pltpu.VMEM_SHARED((NSP, 128), jnp.float32),
        ],
    )
    def k(dst_hbm, ones_hbm, z_hbm, out_hbm, idxd, rows, acc):
        c = lax.axis_index("c")
        s = lax.axis_index("s")
        t = c * NS + s
        pltpu.sync_copy(dst_hbm.at[pl.ds(t * CH, CH)], idxd)
        pltpu.sync_copy(z_hbm, rows)
        for z in range(ZCH):
            pltpu.sync_copy(
                rows, acc.at[pl.ds((s * ZCH + z) * CHUNK, CHUNK)])
        plsc.subcore_barrier()
        pltpu.sync_copy(ones_hbm, rows)

        def body(j, carry):
            pltpu.sync_copy(rows, acc.at[idxd.at[j]], add=True)
            return carry

        lax.fori_loop(0, CH, body, 0)
        plsc.subcore_barrier()
        base = s * WBF

        @pl.when(s < NS - 1)
        def _():
            pltpu.sync_copy(acc.at[pl.ds(base, WBF)],
                            out_hbm.at[c].at[pl.ds(base, WBF)])

        @pl.when(s == NS - 1)
        def _():
            pltpu.sync_copy(acc.at[pl.ds(base, WBL)],
                            out_hbm.at[c].at[pl.ds(base, WBL)])

    return k(dst2d, ones_rows, zrows)


def _dinv_of(cnt0, cnt1):
    return lax.rsqrt(cnt0[:, :1] + cnt1[:, :1] + 1.0)


def _mm_scale(x, W, cnt, bn=2000):
    N, Din = x.shape
    Dout = W.shape[1]

    def body(x_ref, w_ref, cnt_ref, o_ref, dinv_ref):
        dinv = _dinv_of(cnt_ref[0], cnt_ref[1])
        dinv_ref[...] = dinv
        o_ref[...] = jnp.dot(x_ref[...], w_ref[...],
                             preferred_element_type=jnp.float32) * dinv

    return pl.pallas_call(
        body,
        grid=(N // bn,),
        in_specs=[
            pl.BlockSpec((bn, Din), lambda i: (i, 0)),
            pl.BlockSpec((Din, Dout), lambda i: (0, 0)),
            pl.BlockSpec((NC, bn, 128), lambda i: (0, i, 0)),
        ],
        out_specs=[
            pl.BlockSpec((bn, Dout), lambda i: (i, 0)),
            pl.BlockSpec((bn, 1), lambda i: (i, 0)),
        ],
        out_shape=[
            jax.ShapeDtypeStruct((N, Dout), jnp.float32),
            jax.ShapeDtypeStruct((N, 1), jnp.float32),
        ],
    )(x, W, cnt)


def _fuse_mm(s, g, dinv, b, W, bn=2000):
    N, D = g.shape
    Dout = W.shape[1]

    def body(s_ref, g_ref, dinv_ref, b_ref, w_ref, o_ref):
        dinv = dinv_ref[...]
        f = jnp.maximum(
            (s_ref[0] + s_ref[1] + g_ref[...]) * dinv + b_ref[...], 0.0)
        o_ref[...] = jnp.dot(f, w_ref[...],
                             preferred_element_type=jnp.float32) * dinv

    return pl.pallas_call(
        body,
        grid=(N // bn,),
        in_specs=[
            pl.BlockSpec((NC, bn, D), lambda i: (0, i, 0)),
            pl.BlockSpec((bn, D), lambda i: (i, 0)),
            pl.BlockSpec((bn, 1), lambda i: (i, 0)),
            pl.BlockSpec((1, D), lambda i: (0, 0)),
            pl.BlockSpec((D, Dout), lambda i: (0, 0)),
        ],
        out_specs=pl.BlockSpec((bn, Dout), lambda i: (i, 0)),
        out_shape=jax.ShapeDtypeStruct((N, Dout), jnp.float32),
    )(s, g, dinv, b, W)


def _epilogue(s, g, dinv, bmu, bls, bn=2000):
    N, D = g.shape
    H = D // 2

    def body(s_ref, g_ref, dinv_ref, bmu_ref, bls_ref, o1_ref, o2_ref):
        res = (s_ref[0] + s_ref[1] + g_ref[...]) * dinv_ref[...]
        o1_ref[...] = res[:, :H] + bmu_ref[...]
        o2_ref[...] = res[:, H:] + bls_ref[...]

    return pl.pallas_call(
        body,
        grid=(N // bn,),
        in_specs=[
            pl.BlockSpec((NC, bn, D), lambda i: (0, i, 0)),
            pl.BlockSpec((bn, D), lambda i: (i, 0)),
            pl.BlockSpec((bn, 1), lambda i: (i, 0)),
            pl.BlockSpec((1, H), lambda i: (0, 0)),
            pl.BlockSpec((1, H), lambda i: (0, 0)),
        ],
        out_specs=[
            pl.BlockSpec((bn, H), lambda i: (i, 0)),
            pl.BlockSpec((bn, H), lambda i: (i, 0)),
        ],
        out_shape=[
            jax.ShapeDtypeStruct((N, H), jnp.float32),
            jax.ShapeDtypeStruct((N, H), jnp.float32),
        ],
    )(s, g, dinv, bmu, bls)


def kernel(x, edge_index, W1, b1, W2, b2, Wmu, bmu, Wls, bls):
    N, _ = x.shape
    E = edge_index.shape[1]
    assert N % NS == 0
    EP = _round_up(E, NC * NS * CHUNK * 8)
    pad = EP - E
    src = jnp.concatenate(
        [edge_index[0], jnp.arange(pad, dtype=edge_index.dtype) % N])
    dst = jnp.concatenate(
        [edge_index[1], jnp.full((pad,), N, edge_index.dtype)])
    src2d = src.reshape(EP // CHUNK, CHUNK)
    dst2d = dst.reshape(EP // CHUNK, CHUNK)
    src64 = src.reshape(EP // 64, 64)
    dst64 = dst.reshape(EP // 64, 64)
    ones128 = jnp.ones((CHUNK, 128), jnp.float32)
    z128 = jnp.zeros((CHUNK, 128), jnp.float32)

    cnt = _degree(dst2d, ones128, z128, N)

    g1, dinv = _mm_scale(x, W1, cnt)
    s1 = _prop(g1, src64, dst64, z128, N)

    h2 = W2.shape[1]
    W2p = jnp.pad(W2, ((0, 0), (0, 128 - h2)))
    b2p = jnp.pad(b2, (0, 128 - h2))
    g2 = _fuse_mm(s1, g1, dinv, b1.reshape(1, -1), W2p)
    s2 = _prop(g2, src64, dst64, z128, N)

    Wcat = jnp.concatenate([Wmu, Wls], axis=1)
    Wcatp = jnp.pad(Wcat, ((0, 128 - h2), (0, 0)))
    g3 = _fuse_mm(s2, g2, dinv, b2p.reshape(1, -1), Wcatp)
    s3 = _prop(g3, src64, dst64, z128, N)

    mu, logstd = _epilogue(s3, g3, dinv, bmu.reshape(1, -1),
                           bls.reshape(1, -1))
    return mu, logstd
